# Initial kernel scaffold; baseline (speedup 1.0000x reference)
import jax
import jax.numpy as jnp
from jax.experimental import pallas as pl


def kernel(x, embeddings):
    raise NotImplementedError("write your pallas kernel here")


# speedup vs baseline: 1.3528x; 1.3528x over previous
import functools

import jax
import jax.numpy as jnp
from jax import lax
from jax.experimental import pallas as pl
from jax.experimental.pallas import tpu as pltpu
from jax.experimental.pallas import tpu_sc as plsc

EMB = 256
NEMB = 8192
ROWS = 8192
BLK = 256
NBLK = ROWS // BLK
COMMIT = 0.25


def _tc_body(x_ref, e_ref, enc_ref, idx_ref, avg_ref, cent_ref, scal_ref,
             counts_ref, loss_ref):
    i = pl.program_id(0)
    b = pl.program_id(1)
    xb = x_ref[0]
    e = e_ref[...]

    xsq = jnp.sum(xb * xb, axis=1, keepdims=True)
    esq = jnp.sum(e * e, axis=0, keepdims=True)
    m = (2.0 * xb).astype(jnp.bfloat16).astype(jnp.float32)
    conv = jnp.dot(m, e, preferred_element_type=jnp.float32)
    dist = (xsq + esq) - conv

    H = NEMB // 2
    dA = dist[:, :H]
    dB = dist[:, H:]
    minA = jnp.min(dA, axis=1, keepdims=True)
    minB = jnp.min(dB, axis=1, keepdims=True)
    colH = lax.broadcasted_iota(jnp.int32, (BLK, H), 1)
    idxA = jnp.min(jnp.where(dA == minA, colH, NEMB), axis=1)
    idxB = jnp.min(jnp.where(dB == minB, colH + H, NEMB), axis=1)
    thresh = minA.astype(jnp.bfloat16).astype(jnp.float32)
    takeB = minB < thresh
    idx = jnp.where(takeB[:, 0], idxB, idxA)
    minv = jnp.where(takeB, minB, minA)
    col = lax.broadcasted_iota(jnp.int32, (BLK, NEMB), 1)

    onehot = jnp.where(col == idx[:, None], 1.0, 0.0).astype(jnp.float32)
    enc_ref[...] = onehot
    idx_ref[...] = idx.reshape(1, 1, BLK)

    @pl.when(b == 0)
    def _():
        avg_ref[...] = onehot * 0.125

    @pl.when(b != 0)
    def _():
        avg_ref[...] = avg_ref[...] + onehot * 0.125

    @pl.when((i == 0) & (b == 0))
    def _():
        counts_ref[...] = jnp.zeros_like(counts_ref)
        loss_ref[0] = 0.0

    counts_ref[...] = counts_ref[...] + jnp.sum(onehot, axis=0, keepdims=True)
    loss_ref[0] = loss_ref[0] + jnp.sum(minv)

    rb = b * 4 + i
    cent_ref[...] = e_ref[:, pl.ds(rb * BLK, BLK)].T

    @pl.when((i == pl.num_programs(0) - 1) & (b == pl.num_programs(1) - 1))
    def _():
        over = counts_ref[...] * (1.0 / ROWS)
        ent = jnp.sum(over * jnp.log(over + 1e-20))
        ppx = jnp.exp(-ent)
        lossv = (1.0 + COMMIT) * loss_ref[0] / (ROWS * EMB)
        r = lax.broadcasted_iota(jnp.int32, (8, 128), 0)
        c = lax.broadcasted_iota(jnp.int32, (8, 128), 1)
        scal_ref[...] = jnp.where((r == 0) & (c == 0), lossv,
                                  jnp.where((r == 0) & (c == 1), ppx, 0.0))


def _tc_call(x, e):
    return pl.pallas_call(
        _tc_body,
        grid=(ROWS // 1024 // 2, 8),
        in_specs=[
            pl.BlockSpec((1, BLK, EMB), lambda i, b: (b, i, 0)),
            pl.BlockSpec((EMB, NEMB), lambda i, b: (0, 0)),
        ],
        out_specs=[
            pl.BlockSpec((BLK, NEMB), lambda i, b: (b * 4 + i, 0)),
            pl.BlockSpec((1, 1, BLK), lambda i, b: (b * 4 + i, 0, 0)),
            pl.BlockSpec((BLK, NEMB), lambda i, b: (i, 0)),
            pl.BlockSpec((BLK, EMB), lambda i, b: (b * 4 + i, 0)),
            pl.BlockSpec((8, 128), lambda i, b: (0, 0)),
        ],
        out_shape=[
            jax.ShapeDtypeStruct((ROWS, NEMB), jnp.float32),
            jax.ShapeDtypeStruct((NBLK, 1, BLK), jnp.int32),
            jax.ShapeDtypeStruct((1024, NEMB), jnp.float32),
            jax.ShapeDtypeStruct((NEMB, EMB), jnp.float32),
            jax.ShapeDtypeStruct((8, 128), jnp.float32),
        ],
        scratch_shapes=[
            pltpu.VMEM((1, NEMB), jnp.float32),
            pltpu.SMEM((1,), jnp.float32),
        ],
    )(x, e)


def _sc_gather(table, idx):
    NC, NS = 2, 16
    NW = NC * NS
    bpw = ROWS // NW
    mesh = plsc.VectorSubcoreMesh(core_axis_name="c", subcore_axis_name="s")

    @functools.partial(
        pl.kernel, mesh=mesh,
        out_type=jax.ShapeDtypeStruct((ROWS, EMB), jnp.float32),
        scratch_types=[
            pltpu.VMEM((bpw,), jnp.int32),
            pltpu.VMEM((bpw, EMB), jnp.float32),
            pltpu.SemaphoreType.DMA,
        ],
    )
    def k(table_hbm, idx_hbm, out_hbm, idx_v, rows_v, sem):
        wid = lax.axis_index("s") * NC + lax.axis_index("c")
        base = wid * bpw
        pltpu.sync_copy(idx_hbm.at[pl.ds(base, bpw)], idx_v)
        pltpu.async_copy(table_hbm.at[idx_v], rows_v, sem).wait()
        pltpu.sync_copy(rows_v, out_hbm.at[pl.ds(base, bpw)])

    return k(table, idx)


def kernel(x, embeddings):
    enc, idx_blocks, avg_probs, centers, scal = _tc_call(x, embeddings)
    enc_idx = idx_blocks.reshape(ROWS)
    quantized = _sc_gather(centers, enc_idx).reshape(x.shape)
    quantized_st = x + (quantized - x)
    loss = scal[0, 0]
    perplexity = scal[0, 1]
    aux = {
        'encoding': enc,
        'encoding_index': enc_idx,
        'avg_probs': avg_probs,
        'perplexity': perplexity,
        'centers': centers,
    }
    return (quantized_st, loss, aux)


# speedup vs baseline: 1.3832x; 1.0224x over previous
import functools

import jax
import jax.numpy as jnp
from jax import lax
from jax.experimental import pallas as pl
from jax.experimental.pallas import tpu as pltpu
from jax.experimental.pallas import tpu_sc as plsc

EMB = 256
NEMB = 8192
ROWS = 8192
BLK = 256
NBLK = ROWS // BLK
COMMIT = 0.25


def _tc_body(x_ref, e_ref, enc_ref, idx_ref, avg_ref, cent_ref, scal_ref,
             esq_ref, counts_ref, loss_ref):
    i = pl.program_id(0)
    b = pl.program_id(1)
    xb = x_ref[0]
    e = e_ref[...]

    @pl.when((i == 0) & (b == 0))
    def _():
        esq_ref[...] = jnp.sum(e * e, axis=0, keepdims=True)
        counts_ref[...] = jnp.zeros_like(counts_ref)
        loss_ref[0] = 0.0

    xsq = jnp.sum(xb * xb, axis=1, keepdims=True)
    esq = esq_ref[...]
    m = (2.0 * xb).astype(jnp.bfloat16).astype(jnp.float32)
    conv = jnp.dot(m, e, preferred_element_type=jnp.float32)
    dist = (xsq + esq) - conv

    H = NEMB // 2
    dA = dist[:, :H]
    dB = dist[:, H:]
    minA = jnp.min(dA, axis=1, keepdims=True)
    minB = jnp.min(dB, axis=1, keepdims=True)
    colH = lax.broadcasted_iota(jnp.int32, (BLK, H), 1)
    idxA = jnp.min(jnp.where(dA == minA, colH, NEMB), axis=1)
    idxB = jnp.min(jnp.where(dB == minB, colH + H, NEMB), axis=1)
    thresh = minA.astype(jnp.bfloat16).astype(jnp.float32)
    takeB = minB < thresh
    idx = jnp.where(takeB[:, 0], idxB, idxA)
    minv = jnp.where(takeB, minB, minA)
    col = lax.broadcasted_iota(jnp.int32, (BLK, NEMB), 1)

    onehot = jnp.where(col == idx[:, None], 1.0, 0.0).astype(jnp.float32)
    enc_ref[...] = onehot
    idx_ref[...] = idx.reshape(1, 1, BLK)

    @pl.when(b == 0)
    def _():
        avg_ref[...] = onehot

    @pl.when((b != 0) & (b != pl.num_programs(1) - 1))
    def _():
        avg_ref[...] = avg_ref[...] + onehot

    @pl.when(b == pl.num_programs(1) - 1)
    def _():
        raw = avg_ref[...] + onehot
        avg_ref[...] = raw * 0.125
        counts_ref[...] = counts_ref[...] + jnp.sum(raw, axis=0, keepdims=True)

    loss_ref[0] = loss_ref[0] + jnp.sum(minv)

    rb = b * 4 + i
    cent_ref[...] = e_ref[:, pl.ds(rb * BLK, BLK)].T

    @pl.when((i == pl.num_programs(0) - 1) & (b == pl.num_programs(1) - 1))
    def _():
        over = counts_ref[...] * (1.0 / ROWS)
        ent = jnp.sum(over * jnp.log(over + 1e-20))
        ppx = jnp.exp(-ent)
        lossv = (1.0 + COMMIT) * loss_ref[0] / (ROWS * EMB)
        r = lax.broadcasted_iota(jnp.int32, (8, 128), 0)
        c = lax.broadcasted_iota(jnp.int32, (8, 128), 1)
        scal_ref[...] = jnp.where((r == 0) & (c == 0), lossv,
                                  jnp.where((r == 0) & (c == 1), ppx, 0.0))


def _tc_call(x, e):
    return pl.pallas_call(
        _tc_body,
        grid=(ROWS // 1024 // 2, 8),
        in_specs=[
            pl.BlockSpec((1, BLK, EMB), lambda i, b: (b, i, 0)),
            pl.BlockSpec((EMB, NEMB), lambda i, b: (0, 0)),
        ],
        out_specs=[
            pl.BlockSpec((BLK, NEMB), lambda i, b: (b * 4 + i, 0)),
            pl.BlockSpec((1, 1, BLK), lambda i, b: (b * 4 + i, 0, 0)),
            pl.BlockSpec((BLK, NEMB), lambda i, b: (i, 0)),
            pl.BlockSpec((BLK, EMB), lambda i, b: (b * 4 + i, 0)),
            pl.BlockSpec((8, 128), lambda i, b: (0, 0)),
        ],
        out_shape=[
            jax.ShapeDtypeStruct((ROWS, NEMB), jnp.float32),
            jax.ShapeDtypeStruct((NBLK, 1, BLK), jnp.int32),
            jax.ShapeDtypeStruct((1024, NEMB), jnp.float32),
            jax.ShapeDtypeStruct((NEMB, EMB), jnp.float32),
            jax.ShapeDtypeStruct((8, 128), jnp.float32),
        ],
        scratch_shapes=[
            pltpu.VMEM((1, NEMB), jnp.float32),
            pltpu.VMEM((1, NEMB), jnp.float32),
            pltpu.SMEM((1,), jnp.float32),
        ],
    )(x, e)


def _sc_gather(table, idx):
    NC, NS = 2, 16
    NW = NC * NS
    bpw = ROWS // NW
    mesh = plsc.VectorSubcoreMesh(core_axis_name="c", subcore_axis_name="s")

    @functools.partial(
        pl.kernel, mesh=mesh,
        out_type=jax.ShapeDtypeStruct((ROWS, EMB), jnp.float32),
        scratch_types=[
            pltpu.VMEM((bpw,), jnp.int32),
            pltpu.VMEM((bpw, EMB), jnp.float32),
            pltpu.SemaphoreType.DMA,
        ],
    )
    def k(table_hbm, idx_hbm, out_hbm, idx_v, rows_v, sem):
        wid = lax.axis_index("s") * NC + lax.axis_index("c")
        base = wid * bpw
        pltpu.sync_copy(idx_hbm.at[pl.ds(base, bpw)], idx_v)
        pltpu.async_copy(table_hbm.at[idx_v], rows_v, sem).wait()
        pltpu.sync_copy(rows_v, out_hbm.at[pl.ds(base, bpw)])

    return k(table, idx)


def kernel(x, embeddings):
    enc, idx_blocks, avg_probs, centers, scal = _tc_call(x, embeddings)
    enc_idx = idx_blocks.reshape(ROWS)
    quantized = _sc_gather(centers, enc_idx).reshape(x.shape)
    quantized_st = x + (quantized - x)
    loss = scal[0, 0]
    perplexity = scal[0, 1]
    aux = {
        'encoding': enc,
        'encoding_index': enc_idx,
        'avg_probs': avg_probs,
        'perplexity': perplexity,
        'centers': centers,
    }
    return (quantized_st, loss, aux)
